# baseline (speedup 1.0000x reference)
import functools

import jax
import jax.numpy as jnp
from jax import lax
from jax.experimental import pallas as pl
from jax.experimental.pallas import tpu as pltpu
from jax.experimental.pallas import tpu_sc as plsc

N = 10000
D = 128
DH = 64
E = 320000

NC = 2
NS = 16
NPAD = 10240
ROWS_PER_TILE = NPAD // NS

BLK = 512
STEPS = 40
EPT = STEPS * BLK
EPAD = NS * EPT
ER = EPAD // BLK

DEG_STEPS = ER // (NC * NS)

_mesh = plsc.VectorSubcoreMesh(core_axis_name="c", subcore_axis_name="s")


def _deg_kernel_body(dst_hbm, degp_hbm, ones_v, zbuf_v, idx_v, acc_sh):
    cid = lax.axis_index("c")
    sid = lax.axis_index("s")
    wid = sid * NC + cid

    def fill_ones(r, _):
        ones_v[r] = jnp.ones((16,), jnp.float32)
        return 0
    lax.fori_loop(0, BLK, fill_ones, 0, unroll=8)

    def fill_zero(r, _):
        zbuf_v[r] = jnp.zeros((16,), jnp.float32)
        return 0
    lax.fori_loop(0, ROWS_PER_TILE, fill_zero, 0, unroll=8)

    pltpu.sync_copy(zbuf_v, acc_sh.at[pl.ds(sid * ROWS_PER_TILE, ROWS_PER_TILE)])
    plsc.subcore_barrier()

    base = wid * DEG_STEPS
    pltpu.sync_copy(dst_hbm.at[pl.ds(base, DEG_STEPS)], idx_v)

    def step(g, _):
        pltpu.sync_copy(ones_v, acc_sh.at[idx_v.at[g]], add=True)
        return 0
    lax.fori_loop(0, DEG_STEPS, step, 0)

    plsc.subcore_barrier()
    pltpu.sync_copy(acc_sh.at[pl.ds(sid * ROWS_PER_TILE, ROWS_PER_TILE)],
                    degp_hbm.at[cid, pl.ds(sid * ROWS_PER_TILE, ROWS_PER_TILE)])


@functools.partial(
    pl.kernel,
    out_type=jax.ShapeDtypeStruct((NC, NPAD, 16), jnp.float32),
    mesh=_mesh,
    compiler_params=pltpu.CompilerParams(use_tc_tiling_on_sc=False),
    scratch_types=[
        pltpu.VMEM((BLK, 16), jnp.float32),
        pltpu.VMEM((ROWS_PER_TILE, 16), jnp.float32),
        pltpu.VMEM((DEG_STEPS, BLK), jnp.int32),
        pltpu.VMEM_SHARED((NPAD, 16), jnp.float32),
    ],
)
def _deg_kernel(dst_hbm, degp_hbm, ones_v, zbuf_v, idx_v, acc_sh):
    _deg_kernel_body(dst_hbm, degp_hbm, ones_v, zbuf_v, idx_v, acc_sh)


@functools.partial(
    pl.kernel,
    out_type=jax.ShapeDtypeStruct((NC, NPAD, DH), jnp.float32),
    mesh=_mesh,
    compiler_params=pltpu.CompilerParams(use_tc_tiling_on_sc=False),
    scratch_types=[
        [pltpu.VMEM((BLK,), jnp.int32)] * 2,
        [pltpu.VMEM((BLK,), jnp.int32)] * 2,
        [pltpu.VMEM((BLK, DH), jnp.float32)] * 2,
        [pltpu.SemaphoreType.DMA] * 2,
        [pltpu.SemaphoreType.DMA] * 2,
        pltpu.VMEM_SHARED((NPAD, DH), jnp.float32),
    ],
)
def _agg_kernel(z_hbm, src2_hbm, dst_hbm, agg_hbm,
                sidx, didx, rows, gsem, ssem, acc_sh):
    cid = lax.axis_index("c")
    sid = lax.axis_index("s")

    rslice = pl.ds(sid * ROWS_PER_TILE, ROWS_PER_TILE)
    pltpu.sync_copy(z_hbm.at[pl.ds(cid * NPAD + sid * ROWS_PER_TILE,
                                   ROWS_PER_TILE)],
                    acc_sh.at[rslice])
    base = sid * STEPS
    plsc.subcore_barrier()

    def fire_gathers(g, b):
        pltpu.sync_copy(src2_hbm.at[cid, base + g], sidx[b])
        pltpu.sync_copy(dst_hbm.at[base + g], didx[b])
        pltpu.async_copy(z_hbm.at[sidx[b]], rows[b], gsem[b])

    def drain_gathers(g, b):
        pltpu.make_async_copy(z_hbm.at[sidx[b]], rows[b], gsem[b]).wait()

    def fire_scatters(g, b):
        pltpu.async_copy(rows[b], acc_sh.at[didx[b]], ssem[b], add=True)

    def drain_scatters(g, b):
        pltpu.make_async_copy(rows[b], acc_sh.at[didx[b]],
                              ssem[b]).wait()

    fire_gathers(0, 0)

    def body(gg, _):
        g0 = 2 * gg

        @pl.when(gg > 0)
        def _():
            drain_scatters(g0 - 1, 1)
        drain_gathers(g0, 0)
        fire_gathers(g0 + 1, 1)
        fire_scatters(g0, 0)

        drain_gathers(g0 + 1, 1)
        drain_scatters(g0, 0)

        @pl.when(gg < STEPS // 2 - 1)
        def _():
            fire_gathers(g0 + 2, 0)
        fire_scatters(g0 + 1, 1)
        return 0
    lax.fori_loop(0, STEPS // 2, body, 0)
    drain_scatters(STEPS - 1, 1)

    plsc.subcore_barrier()
    pltpu.sync_copy(acc_sh.at[rslice], agg_hbm.at[cid, rslice])


def _dis_from_degp(degp_ref):
    deg = 1.0 + degp_ref[0, :, 0] + degp_ref[1, :, 0]
    return lax.rsqrt(deg)


def _tc_first_body(xp_ref, w_ref, degp_ref, srcr_ref, z_ref, src2_ref):
    dis = _dis_from_degp(degp_ref)
    xw = jnp.dot(xp_ref[...], w_ref[...], preferred_element_type=jnp.float32)
    z = xw * dis[:, None]
    z_ref[pl.ds(0, NPAD), :] = z[:, :DH]
    z_ref[pl.ds(NPAD, NPAD), :] = z[:, DH:]
    src2_ref[0] = srcr_ref[...]
    src2_ref[1] = srcr_ref[...] + NPAD


_tc_first = pl.pallas_call(
    _tc_first_body,
    out_shape=(
        jax.ShapeDtypeStruct((2 * NPAD, DH), jnp.float32),
        jax.ShapeDtypeStruct((NC, ER, BLK), jnp.int32),
    ),
)


def _tc_mid_body(agg_ref, degp_ref, w_ref, b_ref, z_ref):
    dis = _dis_from_degp(degp_ref)
    aggf = jnp.concatenate([agg_ref[0], agg_ref[1]], axis=1)
    h = jnp.maximum(aggf * dis[:, None] + b_ref[...][None, :], 0.0)
    z = jnp.dot(h, w_ref[...], preferred_element_type=jnp.float32) * dis[:, None]
    z_ref[pl.ds(0, NPAD), :] = z[:, :DH]
    z_ref[pl.ds(NPAD, NPAD), :] = z[:, DH:]


_tc_mid = pl.pallas_call(
    _tc_mid_body,
    out_shape=jax.ShapeDtypeStruct((2 * NPAD, DH), jnp.float32),
)


def _tc_final_body(agg_ref, degp_ref, b_ref, out_ref):
    dis = _dis_from_degp(degp_ref)
    aggf = jnp.concatenate([agg_ref[0], agg_ref[1]], axis=1)
    out_ref[...] = aggf * dis[:, None] + b_ref[...][None, :]


_tc_final = pl.pallas_call(
    _tc_final_body,
    out_shape=jax.ShapeDtypeStruct((NPAD, D), jnp.float32),
)


@jax.jit
def kernel(x, edge_index, W1, b1, W2, b2, W3, b3):
    src = edge_index[0]
    dst = edge_index[1]

    pad = EPAD - E
    src_p = jnp.concatenate([src, jnp.zeros((pad,), jnp.int32)])
    dst_p = jnp.concatenate([dst, jnp.full((pad,), NPAD - 1, jnp.int32)])
    src_r = src_p.reshape(ER, BLK)
    dst_r = dst_p.reshape(ER, BLK)

    xp = jnp.zeros((NPAD, D), x.dtype).at[:N].set(x)

    degp = _deg_kernel(dst_r)
    z1, src2 = _tc_first(xp, W1, degp, src_r)
    agg1 = _agg_kernel(z1, src2, dst_r)
    z2 = _tc_mid(agg1, degp, W2, b1)
    agg2 = _agg_kernel(z2, src2, dst_r)
    z3 = _tc_mid(agg2, degp, W3, b2)
    agg3 = _agg_kernel(z3, src2, dst_r)
    out = _tc_final(agg3, degp, b3)
    return out[:N]


# speedup vs baseline: 1.6104x; 1.6104x over previous
import functools

import jax
import jax.numpy as jnp
from jax import lax
from jax.experimental import pallas as pl
from jax.experimental.pallas import tpu as pltpu
from jax.experimental.pallas import tpu_sc as plsc

N = 10000
D = 128
DH = 64
E = 320000

NC = 2
NS = 16
NPAD = 10240
ROWS_PER_TILE = NPAD // NS

BLK = 512
ABLK = 256
EPT = 20480
EPAD = NS * EPT
ER = EPAD // BLK
ERA = EPAD // ABLK
STEPS = EPT // ABLK

DEG_STEPS = ER // (NC * NS)

_mesh = plsc.VectorSubcoreMesh(core_axis_name="c", subcore_axis_name="s")


def _deg_kernel_body(dst_hbm, degp_hbm, ones_v, zbuf_v, idx_v, acc_sh):
    cid = lax.axis_index("c")
    sid = lax.axis_index("s")
    wid = sid * NC + cid

    def fill_ones(r, _):
        ones_v[r] = jnp.ones((16,), jnp.float32)
        return 0
    lax.fori_loop(0, BLK, fill_ones, 0, unroll=8)

    def fill_zero(r, _):
        zbuf_v[r] = jnp.zeros((16,), jnp.float32)
        return 0
    lax.fori_loop(0, ROWS_PER_TILE, fill_zero, 0, unroll=8)

    pltpu.sync_copy(zbuf_v, acc_sh.at[pl.ds(sid * ROWS_PER_TILE, ROWS_PER_TILE)])
    plsc.subcore_barrier()

    base = wid * DEG_STEPS
    pltpu.sync_copy(dst_hbm.at[pl.ds(base, DEG_STEPS)], idx_v)

    def step(g, _):
        pltpu.sync_copy(ones_v, acc_sh.at[idx_v.at[g]], add=True)
        return 0
    lax.fori_loop(0, DEG_STEPS, step, 0)

    plsc.subcore_barrier()
    pltpu.sync_copy(acc_sh.at[pl.ds(sid * ROWS_PER_TILE, ROWS_PER_TILE)],
                    degp_hbm.at[cid, pl.ds(sid * ROWS_PER_TILE, ROWS_PER_TILE)])


@functools.partial(
    pl.kernel,
    out_type=jax.ShapeDtypeStruct((NC, NPAD, 16), jnp.float32),
    mesh=_mesh,
    compiler_params=pltpu.CompilerParams(use_tc_tiling_on_sc=False),
    scratch_types=[
        pltpu.VMEM((BLK, 16), jnp.float32),
        pltpu.VMEM((ROWS_PER_TILE, 16), jnp.float32),
        pltpu.VMEM((DEG_STEPS, BLK), jnp.int32),
        pltpu.VMEM_SHARED((NPAD, 16), jnp.float32),
    ],
)
def _deg_kernel(dst_hbm, degp_hbm, ones_v, zbuf_v, idx_v, acc_sh):
    _deg_kernel_body(dst_hbm, degp_hbm, ones_v, zbuf_v, idx_v, acc_sh)


@functools.partial(
    pl.kernel,
    out_type=jax.ShapeDtypeStruct((NC, NPAD, DH), jnp.float32),
    mesh=_mesh,
    compiler_params=pltpu.CompilerParams(use_tc_tiling_on_sc=False),
    scratch_types=[
        [pltpu.VMEM((ABLK,), jnp.int32)] * 2,
        [pltpu.VMEM((ABLK,), jnp.int32)] * 2,
        [pltpu.VMEM((ABLK, DH), jnp.float32)] * 2,
        [pltpu.SemaphoreType.DMA] * 2,
        [pltpu.SemaphoreType.DMA] * 2,
        pltpu.VMEM_SHARED((NPAD, DH), jnp.float32),
        pltpu.VMEM_SHARED((NPAD, DH), jnp.float32),
    ],
)
def _agg_kernel(z_hbm, src_hbm, dst_hbm, agg_hbm,
                sidx, didx, rows, gsem, ssem, z_sh, acc_sh):
    cid = lax.axis_index("c")
    sid = lax.axis_index("s")

    rslice = pl.ds(sid * ROWS_PER_TILE, ROWS_PER_TILE)
    zsrc = z_hbm.at[pl.ds(cid * NPAD + sid * ROWS_PER_TILE, ROWS_PER_TILE)]
    pltpu.sync_copy(zsrc, z_sh.at[rslice])
    pltpu.sync_copy(zsrc, acc_sh.at[rslice])
    base = sid * STEPS
    plsc.subcore_barrier()

    def fire_gathers(g, b):
        pltpu.sync_copy(src_hbm.at[base + g], sidx[b])
        pltpu.sync_copy(dst_hbm.at[base + g], didx[b])
        pltpu.async_copy(z_sh.at[sidx[b]], rows[b], gsem[b])

    def drain_gathers(g, b):
        pltpu.make_async_copy(z_sh.at[sidx[b]], rows[b], gsem[b]).wait()

    def fire_scatters(g, b):
        pltpu.async_copy(rows[b], acc_sh.at[didx[b]], ssem[b], add=True)

    def drain_scatters(g, b):
        pltpu.make_async_copy(rows[b], acc_sh.at[didx[b]],
                              ssem[b]).wait()

    fire_gathers(0, 0)

    def body(gg, _):
        g0 = 2 * gg

        @pl.when(gg > 0)
        def _():
            drain_scatters(g0 - 1, 1)
        drain_gathers(g0, 0)
        fire_gathers(g0 + 1, 1)
        fire_scatters(g0, 0)

        drain_gathers(g0 + 1, 1)
        drain_scatters(g0, 0)

        @pl.when(gg < STEPS // 2 - 1)
        def _():
            fire_gathers(g0 + 2, 0)
        fire_scatters(g0 + 1, 1)
        return 0
    lax.fori_loop(0, STEPS // 2, body, 0)
    drain_scatters(STEPS - 1, 1)

    plsc.subcore_barrier()
    pltpu.sync_copy(acc_sh.at[rslice], agg_hbm.at[cid, rslice])


def _dis_from_degp(degp_ref):
    deg = 1.0 + degp_ref[0, :, 0] + degp_ref[1, :, 0]
    return lax.rsqrt(deg)


def _tc_first_body(xp_ref, w_ref, degp_ref, z_ref):
    dis = _dis_from_degp(degp_ref)
    xw = jnp.dot(xp_ref[...], w_ref[...], preferred_element_type=jnp.float32)
    z = xw * dis[:, None]
    z_ref[pl.ds(0, NPAD), :] = z[:, :DH]
    z_ref[pl.ds(NPAD, NPAD), :] = z[:, DH:]


_tc_first = pl.pallas_call(
    _tc_first_body,
    out_shape=jax.ShapeDtypeStruct((2 * NPAD, DH), jnp.float32),
)


def _tc_mid_body(agg_ref, degp_ref, w_ref, b_ref, z_ref):
    dis = _dis_from_degp(degp_ref)
    aggf = jnp.concatenate([agg_ref[0], agg_ref[1]], axis=1)
    h = jnp.maximum(aggf * dis[:, None] + b_ref[...][None, :], 0.0)
    z = jnp.dot(h, w_ref[...], preferred_element_type=jnp.float32) * dis[:, None]
    z_ref[pl.ds(0, NPAD), :] = z[:, :DH]
    z_ref[pl.ds(NPAD, NPAD), :] = z[:, DH:]


_tc_mid = pl.pallas_call(
    _tc_mid_body,
    out_shape=jax.ShapeDtypeStruct((2 * NPAD, DH), jnp.float32),
)


def _tc_final_body(agg_ref, degp_ref, b_ref, out_ref):
    dis = _dis_from_degp(degp_ref)
    aggf = jnp.concatenate([agg_ref[0], agg_ref[1]], axis=1)
    out_ref[...] = aggf * dis[:, None] + b_ref[...][None, :]


_tc_final = pl.pallas_call(
    _tc_final_body,
    out_shape=jax.ShapeDtypeStruct((NPAD, D), jnp.float32),
)


@jax.jit
def kernel(x, edge_index, W1, b1, W2, b2, W3, b3):
    src = edge_index[0]
    dst = edge_index[1]

    pad = EPAD - E
    src_p = jnp.concatenate([src, jnp.zeros((pad,), jnp.int32)])
    dst_p = jnp.concatenate([dst, jnp.full((pad,), NPAD - 1, jnp.int32)])
    src_ra = src_p.reshape(ERA, ABLK)
    dst_ra = dst_p.reshape(ERA, ABLK)
    dst_rd = dst_p.reshape(ER, BLK)

    xp = jnp.zeros((NPAD, D), x.dtype).at[:N].set(x)

    degp = _deg_kernel(dst_rd)
    z1 = _tc_first(xp, W1, degp)
    agg1 = _agg_kernel(z1, src_ra, dst_ra)
    z2 = _tc_mid(agg1, degp, W2, b1)
    agg2 = _agg_kernel(z2, src_ra, dst_ra)
    z3 = _tc_mid(agg2, degp, W3, b2)
    agg3 = _agg_kernel(z3, src_ra, dst_ra)
    out = _tc_final(agg3, degp, b3)
    return out[:N]
